# baseline (speedup 1.0000x reference)
import jax
import jax.numpy as jnp
from jax import lax
from jax.experimental import pallas as pl
from jax.experimental.pallas import tpu as pltpu
from jax.experimental.pallas import tpu_sc as plsc

N_NODE = 50000
D = 128
E = 400000

NCORE = 2
NSUB = 16
CHUNK = 12544
NCHUNK = 4
N_PAD = CHUNK * NCHUNK
TRASH = 128
ACC_ROWS = CHUNK + TRASH
ROWS_PER_SUB = 200
E_PAD = ROWS_PER_SUB * 128 * NSUB
IW = 8
NWIN = ROWS_PER_SUB // IW
QN = 5
QWIN = NWIN // QN
G = 32
QMAX = QWIN * 64
QCAP = ((QMAX + G - 1) // G) * G
REG = 16 * QCAP + 16
DRAIN = CHUNK // NSUB
ZROWS = ACC_ROWS // NSUB


def _segsum_body(x_hbm, src_hbm, dst_hbm, z2d_hbm, z1d_hbm, tpck_hbm,
                 s_out, c_out,
                 srcb, dstb, posb, posw, pckall, csrcw, csrcw2, dlw, dlw2,
                 rows_v, rows_v2, zbuf, zcnt, ones_v, cbuf, tpckv, cnt_smem,
                 sem, sem2,
                 acc_sh, cnt_sh, ipck_sh):
    c = lax.axis_index("c")
    s = lax.axis_index("s")
    pltpu.sync_copy(z2d_hbm, zbuf)
    pltpu.sync_copy(z1d_hbm, zcnt)
    pltpu.sync_copy(tpck_hbm, tpckv)
    for v in range(G // 16):
        ones_v[pl.ds(16 * v, 16)] = jnp.ones((16,), jnp.float32)
    iota = lax.iota(jnp.int32, 16)
    lane_base = iota * QCAP
    waste = 16 * QCAP + iota
    rbase = s * REG
    def pbody(p, pcarry):
        chunk = c * (NCHUNK // NCORE) + p
        lo = chunk * CHUNK
        zb = s * ZROWS
        for k in range(ZROWS // 16):
            pltpu.sync_copy(zbuf, acc_sh.at[pl.ds(zb + 16 * k, 16)])
        rem = ZROWS % 16
        if rem:
            pltpu.sync_copy(zbuf.at[pl.ds(0, rem)],
                            acc_sh.at[pl.ds(zb + ZROWS - rem, rem)])
        pltpu.sync_copy(zcnt, cnt_sh.at[pl.ds(s * ZROWS, ZROWS)])
        plsc.subcore_barrier()

        def qbody(q, qcarry):
            def wbody(w, cnt_acc):
                row0 = s * ROWS_PER_SUB + w * IW
                pltpu.sync_copy(src_hbm.at[pl.ds(row0, IW)], srcb)
                pltpu.sync_copy(dst_hbm.at[pl.ds(row0, IW)], dstb)
                for j in range(IW):
                    for v in range(8):
                        d = dstb[j, pl.ds(16 * v, 16)]
                        loc = d - lo
                        m = (loc >= 0) & (loc < CHUNK)
                        pos = jnp.where(m, lane_base + cnt_acc, waste)
                        posb[j, pl.ds(16 * v, 16)] = pos
                        pck = (jnp.left_shift(srcb[j, pl.ds(16 * v, 16)], 14)
                               | (loc & 0x3FFF))
                        srcb[j, pl.ds(16 * v, 16)] = pck
                        cnt_acc = cnt_acc + jnp.where(m, 1, 0)
                for j in range(IW):
                    pltpu.sync_copy(
                        srcb.at[j],
                        ipck_sh.at[pl.ds(rbase, REG)].at[posb.at[j]])
                return cnt_acc

            cnt_acc = lax.fori_loop(QWIN * q, QWIN * q + QWIN, wbody,
                                    jnp.zeros((16,), jnp.int32))

            for lane in range(16):
                cnt_smem[lane] = cnt_acc[lane]

            def unpack(g, lane, sw, dw):
                o = lane * QCAP + G * g
                for t in range(G // 16):
                    pv = pckall[pl.ds(o + 16 * t, 16)]
                    sw[pl.ds(16 * t, 16)] = jnp.right_shift(pv, 14)
                    dw[pl.ds(16 * t, 16)] = pv & 0x3FFF

            def scat(rbuf, dw, psem):
                pltpu.make_async_copy(x_hbm.at[pl.ds(0, G)], rbuf, psem).wait()
                pltpu.sync_copy(rbuf, acc_sh.at[dw], add=True)
                pltpu.sync_copy(ones_v, cnt_sh.at[dw], add=True)

            def lbody(lane, lcarry):
                nl = cnt_smem[lane]
                ngl = lax.div(nl + (G - 1), G)
                tb = ngl * G - G
                for t in range(G // 16):
                    idxv = tb + 16 * t + iota
                    posw[pl.ds(16 * t, 16)] = jnp.where(
                        (idxv >= nl) & (idxv >= 0),
                        lane * QCAP + idxv, waste)
                pltpu.sync_copy(tpckv,
                                ipck_sh.at[pl.ds(rbase, REG)].at[posw])
                return lcarry

            lax.fori_loop(0, 16, lbody, 0)
            pltpu.sync_copy(ipck_sh.at[pl.ds(rbase, 16 * QCAP)], pckall)

            def lbody2(lane, lcarry):
                nl = cnt_smem[lane]
                ngl = lax.div(nl + (G - 1), G)

                def gbody(g, carry):
                    @pl.when(g % 2 == 0)
                    def _():
                        unpack(g, lane, csrcw, dlw)
                        pltpu.async_copy(x_hbm.at[csrcw], rows_v, sem)

                    @pl.when(g % 2 == 1)
                    def _():
                        unpack(g, lane, csrcw2, dlw2)
                        pltpu.async_copy(x_hbm.at[csrcw2], rows_v2, sem2)

                    @pl.when((g > 0) & (g % 2 == 1))
                    def _():
                        scat(rows_v, dlw, sem)

                    @pl.when((g > 0) & (g % 2 == 0))
                    def _():
                        scat(rows_v2, dlw2, sem2)

                    return carry

                lax.fori_loop(0, ngl, gbody, 0)

                @pl.when((ngl > 0) & (ngl % 2 == 1))
                def _():
                    scat(rows_v, dlw, sem)

                @pl.when((ngl > 0) & (ngl % 2 == 0))
                def _():
                    scat(rows_v2, dlw2, sem2)

                return lcarry

            lax.fori_loop(0, 16, lbody2, 0)
            return qcarry

        lax.fori_loop(0, QN, qbody, 0)
        plsc.subcore_barrier()

        pltpu.sync_copy(acc_sh.at[pl.ds(s * DRAIN, DRAIN)],
                        s_out.at[pl.ds(lo + s * DRAIN, DRAIN)])
        pltpu.sync_copy(cnt_sh.at[pl.ds(s * DRAIN, DRAIN)], cbuf)
        pltpu.sync_copy(cbuf, c_out.at[pl.ds(lo + s * DRAIN, DRAIN)])
        plsc.subcore_barrier()
        return pcarry

    lax.fori_loop(0, NCHUNK // NCORE, pbody, 0)


@jax.jit
def _sc_segsum(x, src, dst):
    padn = E_PAD - E
    src_p = jnp.concatenate([src, jnp.zeros((padn,), jnp.int32)]).reshape(-1, 128)
    dst_p = jnp.concatenate([dst, jnp.full((padn,), 2**30, jnp.int32)]).reshape(-1, 128)
    z2d = jnp.zeros((16, D), jnp.float32)
    z1d = jnp.zeros((ZROWS,), jnp.float32)
    tidx = jnp.arange(G, dtype=jnp.int32) % TRASH
    tpck = jnp.left_shift(tidx, 14) | (tidx + CHUNK)
    mesh = plsc.VectorSubcoreMesh(core_axis_name="c", subcore_axis_name="s")
    f = pl.kernel(
        _segsum_body,
        mesh=mesh,
        out_type=[jax.ShapeDtypeStruct((N_PAD, D), jnp.float32),
                  jax.ShapeDtypeStruct((N_PAD,), jnp.float32)],
        scratch_types=[
            pltpu.VMEM((IW, 128), jnp.int32),
            pltpu.VMEM((IW, 128), jnp.int32),
            pltpu.VMEM((IW, 128), jnp.int32),
            pltpu.VMEM((G,), jnp.int32),
            pltpu.VMEM((16 * QCAP,), jnp.int32),
            pltpu.VMEM((G,), jnp.int32),
            pltpu.VMEM((G,), jnp.int32),
            pltpu.VMEM((G,), jnp.int32),
            pltpu.VMEM((G,), jnp.int32),
            pltpu.VMEM((G, D), jnp.float32),
            pltpu.VMEM((G, D), jnp.float32),
            pltpu.VMEM((16, D), jnp.float32),
            pltpu.VMEM((ZROWS,), jnp.float32),
            pltpu.VMEM((G,), jnp.float32),
            pltpu.VMEM((DRAIN,), jnp.float32),
            pltpu.VMEM((G,), jnp.int32),
            pltpu.SMEM((16,), jnp.int32),
            pltpu.SemaphoreType.DMA,
            pltpu.SemaphoreType.DMA,
            pltpu.VMEM_SHARED((ACC_ROWS, D), jnp.float32),
            pltpu.VMEM_SHARED((ACC_ROWS,), jnp.float32),
            pltpu.VMEM_SHARED((NSUB * REG,), jnp.int32),
        ],
    )
    return f(x, src_p, dst_p, z2d, z1d, tpck)


_BM = 1000


def _mm_bias_body(x_ref, w_ref, b_ref, o_ref):
    o_ref[...] = (jnp.dot(x_ref[...], w_ref[...],
                          preferred_element_type=jnp.float32) + b_ref[...])


def _mean_leaky_mm_body(s_ref, c_ref, w_ref, b_ref, o_ref):
    inv = 1.0 / jnp.maximum(c_ref[...], 1.0)
    h = s_ref[...] * inv
    h = jnp.where(h >= 0, h, 0.01 * h)
    o_ref[...] = (jnp.dot(h, w_ref[...],
                          preferred_element_type=jnp.float32) + b_ref[...])


def _mean_body(s_ref, c_ref, o_ref):
    o_ref[...] = s_ref[...] / jnp.maximum(c_ref[...], 1.0)


def _tc_mm_bias(x, w, b):
    n = x.shape[0]
    grid = (n // _BM,)
    return pl.pallas_call(
        _mm_bias_body,
        grid=grid,
        in_specs=[pl.BlockSpec((_BM, D), lambda i: (i, 0)),
                  pl.BlockSpec((D, D), lambda i: (0, 0)),
                  pl.BlockSpec((1, D), lambda i: (0, 0))],
        out_specs=pl.BlockSpec((_BM, D), lambda i: (i, 0)),
        out_shape=jax.ShapeDtypeStruct((n, D), jnp.float32),
    )(x, w, b.reshape(1, D))


def _tc_mean_leaky_mm(s, cnt, w, b):
    grid = (N_NODE // _BM,)
    return pl.pallas_call(
        _mean_leaky_mm_body,
        grid=grid,
        in_specs=[pl.BlockSpec((_BM, D), lambda i: (i, 0)),
                  pl.BlockSpec((_BM, 1), lambda i: (i, 0)),
                  pl.BlockSpec((D, D), lambda i: (0, 0)),
                  pl.BlockSpec((1, D), lambda i: (0, 0))],
        out_specs=pl.BlockSpec((_BM, D), lambda i: (i, 0)),
        out_shape=jax.ShapeDtypeStruct((N_NODE, D), jnp.float32),
    )(s, cnt.reshape(-1, 1), w, b.reshape(1, D))


def _tc_mean(s, cnt):
    grid = (N_NODE // _BM,)
    return pl.pallas_call(
        _mean_body,
        grid=grid,
        in_specs=[pl.BlockSpec((_BM, D), lambda i: (i, 0)),
                  pl.BlockSpec((_BM, 1), lambda i: (i, 0))],
        out_specs=pl.BlockSpec((_BM, D), lambda i: (i, 0)),
        out_shape=jax.ShapeDtypeStruct((N_NODE, D), jnp.float32),
    )(s, cnt.reshape(-1, 1))


def kernel(user_features, website_features, edge_index_clicks,
           edge_index_clicked_by, W1_clicks, b1_clicks, W1_clicked_by,
           b1_clicked_by, W2_clicks, b2_clicks, W2_clicked_by, b2_clicked_by):
    src_c = edge_index_clicks[0].astype(jnp.int32)
    dst_c = edge_index_clicks[1].astype(jnp.int32)
    src_r = edge_index_clicked_by[0].astype(jnp.int32)
    dst_r = edge_index_clicked_by[1].astype(jnp.int32)

    wh_u = _tc_mm_bias(user_features, W1_clicks, b1_clicks)
    s1, c1 = _sc_segsum(wh_u, src_c, dst_c)
    wh_w2 = _tc_mean_leaky_mm(s1, c1, W2_clicked_by, b2_clicked_by)
    s2, c2 = _sc_segsum(wh_w2, src_r, dst_r)
    return _tc_mean(s2, c2)


# speedup vs baseline: 1.1534x; 1.1534x over previous
import jax
import jax.numpy as jnp
from jax import lax
from jax.experimental import pallas as pl
from jax.experimental.pallas import tpu as pltpu
from jax.experimental.pallas import tpu_sc as plsc

N_NODE = 50000
D = 128
E = 400000

NCORE = 2
NSUB = 16
CHUNK = 12544
NCHUNK = 4
N_PAD = CHUNK * NCHUNK
TRASH = 128
ACC_ROWS = CHUNK + TRASH
ROWS_PER_SUB = 200
E_PAD = ROWS_PER_SUB * 128 * NSUB
IW = 8
NWIN = ROWS_PER_SUB // IW
QN = 5
QWIN = NWIN // QN
G = 48
QMAX = QWIN * 64
QCAP = ((QMAX + G - 1) // G) * G
REG = 16 * QCAP + 16
DRAIN = CHUNK // NSUB
ZROWS = ACC_ROWS // NSUB


def _segsum_body(x_hbm, src_hbm, dst_hbm, z2d_hbm, z1d_hbm, tpck_hbm,
                 s_out, c_out,
                 srcb, dstb, posb, pckb, pckl, csrcl, cdstl, dlw,
                 rows_v, rows_v2, zbuf, zcnt, ones_v, cbuf, tpckv, cnt_smem,
                 sem, sem2, sem3,
                 acc_sh, cnt_sh, ipck_sh):
    c = lax.axis_index("c")
    s = lax.axis_index("s")
    pltpu.sync_copy(z2d_hbm, zbuf)
    pltpu.sync_copy(z1d_hbm, zcnt)
    pltpu.sync_copy(tpck_hbm, tpckv)
    for v in range(G // 16):
        ones_v[pl.ds(16 * v, 16)] = jnp.ones((16,), jnp.float32)
    iota = lax.iota(jnp.int32, 16)
    lane_base = iota * QCAP
    waste = 16 * QCAP + iota
    rbase = s * REG
    def pbody(p, pcarry):
        chunk = c * (NCHUNK // NCORE) + p
        lo = chunk * CHUNK
        zb = s * ZROWS
        for k in range(ZROWS // 16):
            pltpu.sync_copy(zbuf, acc_sh.at[pl.ds(zb + 16 * k, 16)])
        rem = ZROWS % 16
        if rem:
            pltpu.sync_copy(zbuf.at[pl.ds(0, rem)],
                            acc_sh.at[pl.ds(zb + ZROWS - rem, rem)])
        pltpu.sync_copy(zcnt, cnt_sh.at[pl.ds(s * ZROWS, ZROWS)])
        plsc.subcore_barrier()

        def qbody(q, qcarry):
            for lane in range(16):
                pltpu.sync_copy(tpckv,
                                ipck_sh.at[pl.ds(rbase + lane * QCAP, QCAP)])

            def wbody(w, cnt_acc):
                row0 = s * ROWS_PER_SUB + w * IW
                pltpu.sync_copy(src_hbm.at[pl.ds(row0, IW)], srcb)
                pltpu.sync_copy(dst_hbm.at[pl.ds(row0, IW)], dstb)
                for j in range(IW):
                    for v in range(8):
                        d = dstb[j, pl.ds(16 * v, 16)]
                        loc = d - lo
                        m = (loc >= 0) & (loc < CHUNK)
                        pos = jnp.where(m, lane_base + cnt_acc, waste)
                        posb[j, pl.ds(16 * v, 16)] = pos
                        pck = (jnp.left_shift(srcb[j, pl.ds(16 * v, 16)], 14)
                               | (loc & 0x3FFF))
                        pckb[j, pl.ds(16 * v, 16)] = pck
                        cnt_acc = cnt_acc + jnp.where(m, 1, 0)
                for j in range(IW):
                    pltpu.async_copy(
                        pckb.at[j],
                        ipck_sh.at[pl.ds(rbase, REG)].at[posb.at[j]], sem3)
                for j in range(IW):
                    pltpu.make_async_copy(
                        pckb.at[j],
                        ipck_sh.at[pl.ds(rbase + 128 * j, 128)], sem3).wait()
                return cnt_acc

            cnt_acc = lax.fori_loop(QWIN * q, QWIN * q + QWIN, wbody,
                                    jnp.zeros((16,), jnp.int32))

            for lane in range(16):
                cnt_smem[lane] = cnt_acc[lane]

            def scat(g, rbuf, psem):
                pltpu.make_async_copy(x_hbm.at[pl.ds(0, G)], rbuf, psem).wait()
                o = G * g
                for t in range(G // 16):
                    dlw[pl.ds(16 * t, 16)] = cdstl[pl.ds(o + 16 * t, 16)]
                pltpu.sync_copy(rbuf, acc_sh.at[dlw], add=True)
                pltpu.sync_copy(ones_v, cnt_sh.at[dlw], add=True)

            def lbody(lane, lcarry):
                nl = cnt_smem[lane]
                ngl = lax.div(nl + (G - 1), G)
                base = rbase + lane * QCAP
                pltpu.sync_copy(ipck_sh.at[pl.ds(base, QCAP)], pckl)
                for k in range(QCAP // 16):
                    pv = pckl[pl.ds(16 * k, 16)]
                    csrcl[pl.ds(16 * k, 16)] = jnp.right_shift(pv, 14)
                    cdstl[pl.ds(16 * k, 16)] = pv & 0x3FFF

                def gbody(g, carry):
                    @pl.when(g % 2 == 0)
                    def _():
                        pltpu.async_copy(x_hbm.at[csrcl.at[pl.ds(G * g, G)]],
                                         rows_v, sem)

                    @pl.when(g % 2 == 1)
                    def _():
                        pltpu.async_copy(x_hbm.at[csrcl.at[pl.ds(G * g, G)]],
                                         rows_v2, sem2)

                    @pl.when((g > 0) & (g % 2 == 1))
                    def _():
                        scat(g - 1, rows_v, sem)

                    @pl.when((g > 0) & (g % 2 == 0))
                    def _():
                        scat(g - 1, rows_v2, sem2)

                    return carry

                lax.fori_loop(0, ngl, gbody, 0)

                @pl.when((ngl > 0) & (ngl % 2 == 1))
                def _():
                    scat(ngl - 1, rows_v, sem)

                @pl.when((ngl > 0) & (ngl % 2 == 0))
                def _():
                    scat(ngl - 1, rows_v2, sem2)

                return lcarry

            lax.fori_loop(0, 16, lbody, 0)
            return qcarry

        lax.fori_loop(0, QN, qbody, 0)
        plsc.subcore_barrier()

        pltpu.sync_copy(acc_sh.at[pl.ds(s * DRAIN, DRAIN)],
                        s_out.at[pl.ds(lo + s * DRAIN, DRAIN)])
        pltpu.sync_copy(cnt_sh.at[pl.ds(s * DRAIN, DRAIN)], cbuf)
        pltpu.sync_copy(cbuf, c_out.at[pl.ds(lo + s * DRAIN, DRAIN)])
        plsc.subcore_barrier()
        return pcarry

    lax.fori_loop(0, NCHUNK // NCORE, pbody, 0)


@jax.jit
def _sc_segsum(x, src, dst):
    padn = E_PAD - E
    src_p = jnp.concatenate([src, jnp.zeros((padn,), jnp.int32)]).reshape(-1, 128)
    dst_p = jnp.concatenate([dst, jnp.full((padn,), 2**30, jnp.int32)]).reshape(-1, 128)
    z2d = jnp.zeros((16, D), jnp.float32)
    z1d = jnp.zeros((ZROWS,), jnp.float32)
    tidx = jnp.arange(QCAP, dtype=jnp.int32) % 128
    tpck = jnp.left_shift(tidx, 14) | (tidx + CHUNK)
    mesh = plsc.VectorSubcoreMesh(core_axis_name="c", subcore_axis_name="s")
    f = pl.kernel(
        _segsum_body,
        mesh=mesh,
        out_type=[jax.ShapeDtypeStruct((N_PAD, D), jnp.float32),
                  jax.ShapeDtypeStruct((N_PAD,), jnp.float32)],
        scratch_types=[
            pltpu.VMEM((IW, 128), jnp.int32),
            pltpu.VMEM((IW, 128), jnp.int32),
            pltpu.VMEM((IW, 128), jnp.int32),
            pltpu.VMEM((IW, 128), jnp.int32),
            pltpu.VMEM((QCAP,), jnp.int32),
            pltpu.VMEM((QCAP,), jnp.int32),
            pltpu.VMEM((QCAP,), jnp.int32),
            pltpu.VMEM((G,), jnp.int32),
            pltpu.VMEM((G, D), jnp.float32),
            pltpu.VMEM((G, D), jnp.float32),
            pltpu.VMEM((16, D), jnp.float32),
            pltpu.VMEM((ZROWS,), jnp.float32),
            pltpu.VMEM((G,), jnp.float32),
            pltpu.VMEM((DRAIN,), jnp.float32),
            pltpu.VMEM((QCAP,), jnp.int32),
            pltpu.SMEM((16,), jnp.int32),
            pltpu.SemaphoreType.DMA,
            pltpu.SemaphoreType.DMA,
            pltpu.SemaphoreType.DMA,
            pltpu.VMEM_SHARED((ACC_ROWS, D), jnp.float32),
            pltpu.VMEM_SHARED((ACC_ROWS,), jnp.float32),
            pltpu.VMEM_SHARED((NSUB * REG,), jnp.int32),
        ],
    )
    return f(x, src_p, dst_p, z2d, z1d, tpck)


_BM = 1000


def _mm_bias_body(x_ref, w_ref, b_ref, o_ref):
    o_ref[...] = (jnp.dot(x_ref[...], w_ref[...],
                          preferred_element_type=jnp.float32) + b_ref[...])


def _mean_leaky_mm_body(s_ref, c_ref, w_ref, b_ref, o_ref):
    inv = 1.0 / jnp.maximum(c_ref[...], 1.0)
    h = s_ref[...] * inv
    h = jnp.where(h >= 0, h, 0.01 * h)
    o_ref[...] = (jnp.dot(h, w_ref[...],
                          preferred_element_type=jnp.float32) + b_ref[...])


def _mean_body(s_ref, c_ref, o_ref):
    o_ref[...] = s_ref[...] / jnp.maximum(c_ref[...], 1.0)


def _tc_mm_bias(x, w, b):
    n = x.shape[0]
    grid = (n // _BM,)
    return pl.pallas_call(
        _mm_bias_body,
        grid=grid,
        in_specs=[pl.BlockSpec((_BM, D), lambda i: (i, 0)),
                  pl.BlockSpec((D, D), lambda i: (0, 0)),
                  pl.BlockSpec((1, D), lambda i: (0, 0))],
        out_specs=pl.BlockSpec((_BM, D), lambda i: (i, 0)),
        out_shape=jax.ShapeDtypeStruct((n, D), jnp.float32),
    )(x, w, b.reshape(1, D))


def _tc_mean_leaky_mm(s, cnt, w, b):
    grid = (N_NODE // _BM,)
    return pl.pallas_call(
        _mean_leaky_mm_body,
        grid=grid,
        in_specs=[pl.BlockSpec((_BM, D), lambda i: (i, 0)),
                  pl.BlockSpec((_BM, 1), lambda i: (i, 0)),
                  pl.BlockSpec((D, D), lambda i: (0, 0)),
                  pl.BlockSpec((1, D), lambda i: (0, 0))],
        out_specs=pl.BlockSpec((_BM, D), lambda i: (i, 0)),
        out_shape=jax.ShapeDtypeStruct((N_NODE, D), jnp.float32),
    )(s, cnt.reshape(-1, 1), w, b.reshape(1, D))


def _tc_mean(s, cnt):
    grid = (N_NODE // _BM,)
    return pl.pallas_call(
        _mean_body,
        grid=grid,
        in_specs=[pl.BlockSpec((_BM, D), lambda i: (i, 0)),
                  pl.BlockSpec((_BM, 1), lambda i: (i, 0))],
        out_specs=pl.BlockSpec((_BM, D), lambda i: (i, 0)),
        out_shape=jax.ShapeDtypeStruct((N_NODE, D), jnp.float32),
    )(s, cnt.reshape(-1, 1))


def kernel(user_features, website_features, edge_index_clicks,
           edge_index_clicked_by, W1_clicks, b1_clicks, W1_clicked_by,
           b1_clicked_by, W2_clicks, b2_clicks, W2_clicked_by, b2_clicked_by):
    src_c = edge_index_clicks[0].astype(jnp.int32)
    dst_c = edge_index_clicks[1].astype(jnp.int32)
    src_r = edge_index_clicked_by[0].astype(jnp.int32)
    dst_r = edge_index_clicked_by[1].astype(jnp.int32)

    wh_u = _tc_mm_bias(user_features, W1_clicks, b1_clicks)
    s1, c1 = _sc_segsum(wh_u, src_c, dst_c)
    wh_w2 = _tc_mean_leaky_mm(s1, c1, W2_clicked_by, b2_clicked_by)
    s2, c2 = _sc_segsum(wh_w2, src_r, dst_r)
    return _tc_mean(s2, c2)


# speedup vs baseline: 1.1637x; 1.0089x over previous
import jax
import jax.numpy as jnp
from jax import lax
from jax.experimental import pallas as pl
from jax.experimental.pallas import tpu as pltpu
from jax.experimental.pallas import tpu_sc as plsc

N_NODE = 50000
D = 128
E = 400000

NCORE = 2
NSUB = 16
CHUNK = 12544
NCHUNK = 4
N_PAD = CHUNK * NCHUNK
TRASH = 128
ACC_ROWS = CHUNK + TRASH
ROWS_PER_SUB = 200
E_PAD = ROWS_PER_SUB * 128 * NSUB
IW = 8
NWIN = ROWS_PER_SUB // IW
QN = 5
QWIN = NWIN // QN
G = 48
QMAX = QWIN * 64
QCAP = ((QMAX + G - 1) // G) * G
REG = 16 * QCAP + 16
DRAIN = CHUNK // NSUB
ZROWS = ACC_ROWS // NSUB


def _segsum_body(x_hbm, src_hbm, dst_hbm, z2d_hbm, z1d_hbm, tpck_hbm,
                 s_out, c_out,
                 srcb, dstb, posb, pckb, pckl, csrcl, cdstl, dlw,
                 rows_v, rows_v2, zbuf, zcnt, ones_v, cbuf, tpckv, cnt_smem,
                 sem, sem2, sem3,
                 acc_sh, cnt_sh, ipck_sh):
    c = lax.axis_index("c")
    s = lax.axis_index("s")
    pltpu.sync_copy(z2d_hbm, zbuf)
    pltpu.sync_copy(z1d_hbm, zcnt)
    pltpu.sync_copy(tpck_hbm, tpckv)
    for v in range(G // 16):
        ones_v[pl.ds(16 * v, 16)] = jnp.ones((16,), jnp.float32)
    iota = lax.iota(jnp.int32, 16)
    lane_base = iota * QCAP
    waste = 16 * QCAP + iota
    rbase = s * REG
    def pbody(p, pcarry):
        chunk = c * (NCHUNK // NCORE) + p
        lo = chunk * CHUNK
        zb = s * ZROWS
        for k in range(ZROWS // 16):
            pltpu.async_copy(zbuf, acc_sh.at[pl.ds(zb + 16 * k, 16)], sem3)
        rem = ZROWS % 16
        if rem:
            pltpu.async_copy(zbuf.at[pl.ds(0, rem)],
                             acc_sh.at[pl.ds(zb + ZROWS - rem, rem)], sem3)
        pltpu.async_copy(zcnt, cnt_sh.at[pl.ds(s * ZROWS, ZROWS)], sem3)
        for k in range(ZROWS // 16):
            pltpu.make_async_copy(zbuf, acc_sh.at[pl.ds(zb + 16 * k, 16)],
                                  sem3).wait()
        if rem:
            pltpu.make_async_copy(zbuf.at[pl.ds(0, rem)],
                                  acc_sh.at[pl.ds(zb + ZROWS - rem, rem)],
                                  sem3).wait()
        pltpu.make_async_copy(zcnt, cnt_sh.at[pl.ds(s * ZROWS, ZROWS)],
                              sem3).wait()
        plsc.subcore_barrier()

        def qbody(q, qcarry):
            for lane in range(16):
                pltpu.sync_copy(tpckv,
                                ipck_sh.at[pl.ds(rbase + lane * QCAP, QCAP)])

            def wbody(w, cnt_acc):
                row0 = s * ROWS_PER_SUB + w * IW
                pltpu.sync_copy(src_hbm.at[pl.ds(row0, IW)], srcb)
                pltpu.sync_copy(dst_hbm.at[pl.ds(row0, IW)], dstb)
                for j in range(IW):
                    for v in range(8):
                        d = dstb[j, pl.ds(16 * v, 16)]
                        loc = d - lo
                        m = (loc >= 0) & (loc < CHUNK)
                        pos = jnp.where(m, lane_base + cnt_acc, waste)
                        posb[j, pl.ds(16 * v, 16)] = pos
                        pck = (jnp.left_shift(srcb[j, pl.ds(16 * v, 16)], 14)
                               | (loc & 0x3FFF))
                        pckb[j, pl.ds(16 * v, 16)] = pck
                        cnt_acc = cnt_acc + jnp.where(m, 1, 0)
                for j in range(IW):
                    pltpu.async_copy(
                        pckb.at[j],
                        ipck_sh.at[pl.ds(rbase, REG)].at[posb.at[j]], sem3)
                for j in range(IW):
                    pltpu.make_async_copy(
                        pckb.at[j],
                        ipck_sh.at[pl.ds(rbase + 128 * j, 128)], sem3).wait()
                return cnt_acc

            cnt_acc = lax.fori_loop(QWIN * q, QWIN * q + QWIN, wbody,
                                    jnp.zeros((16,), jnp.int32))

            for lane in range(16):
                cnt_smem[lane] = cnt_acc[lane]

            def scat(g, rbuf, psem):
                pltpu.make_async_copy(x_hbm.at[pl.ds(0, G)], rbuf, psem).wait()
                o = G * g
                for t in range(G // 16):
                    dlw[pl.ds(16 * t, 16)] = cdstl[pl.ds(o + 16 * t, 16)]
                pltpu.sync_copy(rbuf, acc_sh.at[dlw], add=True)
                pltpu.sync_copy(ones_v, cnt_sh.at[dlw], add=True)

            def lbody(lane, lcarry):
                nl = cnt_smem[lane]
                ngl = lax.div(nl + (G - 1), G)
                base = rbase + lane * QCAP
                pltpu.sync_copy(ipck_sh.at[pl.ds(base, QCAP)], pckl)
                for k in range(QCAP // 16):
                    pv = pckl[pl.ds(16 * k, 16)]
                    csrcl[pl.ds(16 * k, 16)] = jnp.right_shift(pv, 14)
                    cdstl[pl.ds(16 * k, 16)] = pv & 0x3FFF

                def gbody(g, carry):
                    @pl.when(g % 2 == 0)
                    def _():
                        pltpu.async_copy(x_hbm.at[csrcl.at[pl.ds(G * g, G)]],
                                         rows_v, sem)

                    @pl.when(g % 2 == 1)
                    def _():
                        pltpu.async_copy(x_hbm.at[csrcl.at[pl.ds(G * g, G)]],
                                         rows_v2, sem2)

                    @pl.when((g > 0) & (g % 2 == 1))
                    def _():
                        scat(g - 1, rows_v, sem)

                    @pl.when((g > 0) & (g % 2 == 0))
                    def _():
                        scat(g - 1, rows_v2, sem2)

                    return carry

                lax.fori_loop(0, ngl, gbody, 0)

                @pl.when((ngl > 0) & (ngl % 2 == 1))
                def _():
                    scat(ngl - 1, rows_v, sem)

                @pl.when((ngl > 0) & (ngl % 2 == 0))
                def _():
                    scat(ngl - 1, rows_v2, sem2)

                return lcarry

            lax.fori_loop(0, 16, lbody, 0)
            return qcarry

        lax.fori_loop(0, QN, qbody, 0)
        plsc.subcore_barrier()

        pltpu.sync_copy(acc_sh.at[pl.ds(s * DRAIN, DRAIN)],
                        s_out.at[pl.ds(lo + s * DRAIN, DRAIN)])
        pltpu.sync_copy(cnt_sh.at[pl.ds(s * DRAIN, DRAIN)], cbuf)
        pltpu.sync_copy(cbuf, c_out.at[pl.ds(lo + s * DRAIN, DRAIN)])
        plsc.subcore_barrier()
        return pcarry

    lax.fori_loop(0, NCHUNK // NCORE, pbody, 0)


@jax.jit
def _sc_segsum(x, src, dst):
    padn = E_PAD - E
    src_p = jnp.concatenate([src, jnp.zeros((padn,), jnp.int32)]).reshape(-1, 128)
    dst_p = jnp.concatenate([dst, jnp.full((padn,), 2**30, jnp.int32)]).reshape(-1, 128)
    z2d = jnp.zeros((16, D), jnp.float32)
    z1d = jnp.zeros((ZROWS,), jnp.float32)
    tidx = jnp.arange(QCAP, dtype=jnp.int32) % 128
    tpck = jnp.left_shift(tidx, 14) | (tidx + CHUNK)
    mesh = plsc.VectorSubcoreMesh(core_axis_name="c", subcore_axis_name="s")
    f = pl.kernel(
        _segsum_body,
        mesh=mesh,
        out_type=[jax.ShapeDtypeStruct((N_PAD, D), jnp.float32),
                  jax.ShapeDtypeStruct((N_PAD,), jnp.float32)],
        scratch_types=[
            pltpu.VMEM((IW, 128), jnp.int32),
            pltpu.VMEM((IW, 128), jnp.int32),
            pltpu.VMEM((IW, 128), jnp.int32),
            pltpu.VMEM((IW, 128), jnp.int32),
            pltpu.VMEM((QCAP,), jnp.int32),
            pltpu.VMEM((QCAP,), jnp.int32),
            pltpu.VMEM((QCAP,), jnp.int32),
            pltpu.VMEM((G,), jnp.int32),
            pltpu.VMEM((G, D), jnp.float32),
            pltpu.VMEM((G, D), jnp.float32),
            pltpu.VMEM((16, D), jnp.float32),
            pltpu.VMEM((ZROWS,), jnp.float32),
            pltpu.VMEM((G,), jnp.float32),
            pltpu.VMEM((DRAIN,), jnp.float32),
            pltpu.VMEM((QCAP,), jnp.int32),
            pltpu.SMEM((16,), jnp.int32),
            pltpu.SemaphoreType.DMA,
            pltpu.SemaphoreType.DMA,
            pltpu.SemaphoreType.DMA,
            pltpu.VMEM_SHARED((ACC_ROWS, D), jnp.float32),
            pltpu.VMEM_SHARED((ACC_ROWS,), jnp.float32),
            pltpu.VMEM_SHARED((NSUB * REG,), jnp.int32),
        ],
    )
    return f(x, src_p, dst_p, z2d, z1d, tpck)


_BM = 1000


def _mm_bias_body(x_ref, w_ref, b_ref, o_ref):
    o_ref[...] = (jnp.dot(x_ref[...], w_ref[...],
                          preferred_element_type=jnp.float32) + b_ref[...])


def _mean_leaky_mm_body(s_ref, c_ref, w_ref, b_ref, o_ref):
    inv = 1.0 / jnp.maximum(c_ref[...], 1.0)
    h = s_ref[...] * inv
    h = jnp.where(h >= 0, h, 0.01 * h)
    o_ref[...] = (jnp.dot(h, w_ref[...],
                          preferred_element_type=jnp.float32) + b_ref[...])


def _mean_body(s_ref, c_ref, o_ref):
    o_ref[...] = s_ref[...] / jnp.maximum(c_ref[...], 1.0)


def _tc_mm_bias(x, w, b):
    n = x.shape[0]
    grid = (n // _BM,)
    return pl.pallas_call(
        _mm_bias_body,
        grid=grid,
        in_specs=[pl.BlockSpec((_BM, D), lambda i: (i, 0)),
                  pl.BlockSpec((D, D), lambda i: (0, 0)),
                  pl.BlockSpec((1, D), lambda i: (0, 0))],
        out_specs=pl.BlockSpec((_BM, D), lambda i: (i, 0)),
        out_shape=jax.ShapeDtypeStruct((n, D), jnp.float32),
    )(x, w, b.reshape(1, D))


def _tc_mean_leaky_mm(s, cnt, w, b):
    grid = (N_NODE // _BM,)
    return pl.pallas_call(
        _mean_leaky_mm_body,
        grid=grid,
        in_specs=[pl.BlockSpec((_BM, D), lambda i: (i, 0)),
                  pl.BlockSpec((_BM, 1), lambda i: (i, 0)),
                  pl.BlockSpec((D, D), lambda i: (0, 0)),
                  pl.BlockSpec((1, D), lambda i: (0, 0))],
        out_specs=pl.BlockSpec((_BM, D), lambda i: (i, 0)),
        out_shape=jax.ShapeDtypeStruct((N_NODE, D), jnp.float32),
    )(s, cnt.reshape(-1, 1), w, b.reshape(1, D))


def _tc_mean(s, cnt):
    grid = (N_NODE // _BM,)
    return pl.pallas_call(
        _mean_body,
        grid=grid,
        in_specs=[pl.BlockSpec((_BM, D), lambda i: (i, 0)),
                  pl.BlockSpec((_BM, 1), lambda i: (i, 0))],
        out_specs=pl.BlockSpec((_BM, D), lambda i: (i, 0)),
        out_shape=jax.ShapeDtypeStruct((N_NODE, D), jnp.float32),
    )(s, cnt.reshape(-1, 1))


def kernel(user_features, website_features, edge_index_clicks,
           edge_index_clicked_by, W1_clicks, b1_clicks, W1_clicked_by,
           b1_clicked_by, W2_clicks, b2_clicks, W2_clicked_by, b2_clicked_by):
    src_c = edge_index_clicks[0].astype(jnp.int32)
    dst_c = edge_index_clicks[1].astype(jnp.int32)
    src_r = edge_index_clicked_by[0].astype(jnp.int32)
    dst_r = edge_index_clicked_by[1].astype(jnp.int32)

    wh_u = _tc_mm_bias(user_features, W1_clicks, b1_clicks)
    s1, c1 = _sc_segsum(wh_u, src_c, dst_c)
    wh_w2 = _tc_mean_leaky_mm(s1, c1, W2_clicked_by, b2_clicked_by)
    s2, c2 = _sc_segsum(wh_w2, src_r, dst_r)
    return _tc_mean(s2, c2)
